# baseline (speedup 1.0000x reference)
import functools

import jax
import jax.numpy as jnp
from jax import lax
from jax.experimental import pallas as pl
from jax.experimental.pallas import tpu as pltpu
from jax.experimental.pallas import tpu_sc as plsc

E = 8
TOP_K = 2
D = 1024
H = 4096
N = 2048
BLK = 128
G = (TOP_K * N) // BLK + E
S = G * BLK
HT = 512
NH = H // HT

NEG_BIG = -1e30


def _routing_body(x_ref, wg_ref, slot0_ref, slot1_ref, tok_ref, ws_ref, be_ref):
    x = x_ref[...]
    wg = wg_ref[...]
    logits = lax.dot_general(x, wg, (((1,), (1,)), ((), ())),
                             preferred_element_type=jnp.float32)
    m = jnp.max(logits, axis=-1, keepdims=True)
    ex = jnp.exp(logits - m)
    gw = ex / jnp.sum(ex, axis=-1, keepdims=True)

    iota8 = lax.broadcasted_iota(jnp.int32, (N, E), 1)
    m0 = jnp.max(gw, axis=-1, keepdims=True)
    e0 = jnp.min(jnp.where(gw == m0, iota8, E), axis=-1)
    gw1 = jnp.where(iota8 == e0[:, None], NEG_BIG, gw)
    m1 = jnp.max(gw1, axis=-1, keepdims=True)
    e1 = jnp.min(jnp.where(gw1 == m1, iota8, E), axis=-1)
    w0 = m0[:, 0]
    w1 = m1[:, 0]
    denom = w0 + w1 + 1e-9
    w0 = w0 / denom
    w1 = w1 / denom

    oh0 = (iota8 == e0[:, None]).astype(jnp.float32)
    oh1 = (iota8 == e1[:, None]).astype(jnp.float32)

    ltri = (lax.broadcasted_iota(jnp.int32, (BLK, BLK), 0)
            > lax.broadcasted_iota(jnp.int32, (BLK, BLK), 1)).astype(jnp.float32)
    nchunks = N // BLK

    def excl_cumsum(oh, start):
        parts = []
        carry = start
        for c in range(nchunks):
            ch = oh[c * BLK:(c + 1) * BLK, :]
            parts.append(carry + lax.dot_general(
                ltri, ch, (((1,), (0,)), ((), ())),
                preferred_element_type=jnp.float32))
            carry = carry + jnp.sum(ch, axis=0, keepdims=True)
        return jnp.concatenate(parts, axis=0), carry

    zero_row = jnp.zeros((1, E), jnp.float32)
    rank0, c0 = excl_cumsum(oh0, zero_row)
    rank1, ctot = excl_cumsum(oh1, c0)

    counts = ctot[0]
    nblk = jnp.floor((counts + (BLK - 1)) / BLK)
    ltri_e = (lax.broadcasted_iota(jnp.int32, (E, E), 0)
              > lax.broadcasted_iota(jnp.int32, (E, E), 1)).astype(jnp.float32)
    padded_off = BLK * jnp.dot(ltri_e, nblk,
                               preferred_element_type=jnp.float32)

    slot0f = jnp.sum((padded_off[None, :] + rank0) * oh0, axis=-1)
    slot1f = jnp.sum((padded_off[None, :] + rank1) * oh1, axis=-1)
    slot0_ref[...] = slot0f.astype(jnp.int32)
    slot1_ref[...] = slot1f.astype(jnp.int32)

    gidx = (lax.broadcasted_iota(jnp.int32, (G, E - 1), 0) * BLK).astype(jnp.float32)
    starts = padded_off[None, 1:]
    be_ref[...] = jnp.sum((gidx >= starts).astype(jnp.int32), axis=-1)

    tokf = lax.broadcasted_iota(jnp.int32, (N, 1), 0).astype(jnp.float32)
    s0 = slot0f[None, :]
    s1 = slot1f[None, :]
    rows0 = lax.broadcasted_iota(jnp.int32, (BLK, 1), 0).astype(jnp.float32)
    for b in range(G):
        rows = rows0 + b * BLK
        m0b = (rows == s0).astype(jnp.float32)
        m1b = (rows == s1).astype(jnp.float32)
        tok_b = (jnp.dot(m0b, tokf, preferred_element_type=jnp.float32)
                 + jnp.dot(m1b, tokf, preferred_element_type=jnp.float32))
        ws_b = (jnp.dot(m0b, w0[:, None], preferred_element_type=jnp.float32)
                + jnp.dot(m1b, w1[:, None], preferred_element_type=jnp.float32))
        tok_ref[b * BLK:(b + 1) * BLK, :] = tok_b.astype(jnp.int32)
        ws_ref[b * BLK:(b + 1) * BLK, :] = ws_b


def _routing(x_flat, Wg):
    return pl.pallas_call(
        _routing_body,
        out_shape=[
            jax.ShapeDtypeStruct((N,), jnp.int32),
            jax.ShapeDtypeStruct((N,), jnp.int32),
            jax.ShapeDtypeStruct((S, 1), jnp.int32),
            jax.ShapeDtypeStruct((S, 1), jnp.float32),
            jax.ShapeDtypeStruct((G,), jnp.int32),
        ],
    )(x_flat, Wg)


_NC = 2
_NS = 16
_NW = _NC * _NS
_GCH = 32


def _sc_gather_body(tok_hbm, x_hbm, out_hbm, idx_v, rows_v, sem):
    wid = lax.axis_index("s") * _NC + lax.axis_index("c")
    per_w = S // _NW
    base = wid * per_w
    for c in range(per_w // _GCH):
        off = base + c * _GCH
        pltpu.sync_copy(tok_hbm.at[pl.ds(off, _GCH)], idx_v)
        pltpu.async_copy(x_hbm.at[idx_v], rows_v, sem).wait()
        pltpu.sync_copy(rows_v, out_hbm.at[pl.ds(off, _GCH)])


def _sc_gather(sorted_tok, x_flat):
    mesh = plsc.VectorSubcoreMesh(core_axis_name="c", subcore_axis_name="s")
    fn = functools.partial(
        pl.kernel, mesh=mesh,
        out_type=jax.ShapeDtypeStruct((S, D), jnp.float32),
        scratch_types=[
            pltpu.VMEM((_GCH,), jnp.int32),
            pltpu.VMEM((_GCH, D), jnp.float32),
            pltpu.SemaphoreType.DMA,
        ],
    )(_sc_gather_body)
    return fn(sorted_tok, x_flat)


def _ffn_body(be_ref, x_ref, w1_ref, w2_ref, ws_ref, out_ref):
    h = pl.program_id(1)
    xb = x_ref[...]
    h1 = lax.dot_general(xb, w1_ref[0], (((1,), (1,)), ((), ())),
                         preferred_element_type=jnp.float32)
    h1 = jnp.maximum(h1, 0.0)
    contrib = lax.dot_general(h1, w2_ref[0], (((1,), (1,)), ((), ())),
                              preferred_element_type=jnp.float32)

    @pl.when(h == 0)
    def _():
        out_ref[...] = contrib

    @pl.when(h > 0)
    def _():
        out_ref[...] = out_ref[...] + contrib

    @pl.when(h == NH - 1)
    def _():
        out_ref[...] = out_ref[...] * ws_ref[...]


def _ffn(block_expert, x_sorted, W1, W2, w_sorted):
    grid_spec = pltpu.PrefetchScalarGridSpec(
        num_scalar_prefetch=1,
        grid=(G, NH),
        in_specs=[
            pl.BlockSpec((BLK, D), lambda g, h, be: (g, 0)),
            pl.BlockSpec((1, HT, D), lambda g, h, be: (be[g], h, 0)),
            pl.BlockSpec((1, D, HT), lambda g, h, be: (be[g], 0, h)),
            pl.BlockSpec((BLK, 1), lambda g, h, be: (g, 0)),
        ],
        out_specs=pl.BlockSpec((BLK, D), lambda g, h, be: (g, 0)),
    )
    return pl.pallas_call(
        _ffn_body,
        grid_spec=grid_spec,
        out_shape=jax.ShapeDtypeStruct((S, D), jnp.float32),
    )(block_expert, x_sorted, W1, W2, w_sorted)


_CCH = 32


def _sc_combine_body(s0_hbm, s1_hbm, os_hbm, out_hbm, idx_v, a_v, b_v, sem):
    wid = lax.axis_index("s") * _NC + lax.axis_index("c")
    per_w = N // _NW
    base = wid * per_w
    for c in range(per_w // _CCH):
        off = base + c * _CCH
        pltpu.sync_copy(s0_hbm.at[pl.ds(off, _CCH)], idx_v)
        pltpu.async_copy(os_hbm.at[idx_v], a_v, sem).wait()
        pltpu.sync_copy(s1_hbm.at[pl.ds(off, _CCH)], idx_v)
        pltpu.async_copy(os_hbm.at[idx_v], b_v, sem).wait()

        def add_row(r, carry):
            for j in range(D // 16):
                a_v[r, pl.ds(j * 16, 16)] = (a_v[r, pl.ds(j * 16, 16)]
                                             + b_v[r, pl.ds(j * 16, 16)])
            return carry

        lax.fori_loop(0, _CCH, add_row, 0)
        pltpu.sync_copy(a_v, out_hbm.at[pl.ds(off, _CCH)])


def _sc_combine(slot0, slot1, out_sorted):
    mesh = plsc.VectorSubcoreMesh(core_axis_name="c", subcore_axis_name="s")
    fn = functools.partial(
        pl.kernel, mesh=mesh,
        out_type=jax.ShapeDtypeStruct((N, D), jnp.float32),
        scratch_types=[
            pltpu.VMEM((_CCH,), jnp.int32),
            pltpu.VMEM((_CCH, D), jnp.float32),
            pltpu.VMEM((_CCH, D), jnp.float32),
            pltpu.SemaphoreType.DMA,
        ],
    )(_sc_combine_body)
    return fn(slot0, slot1, out_sorted)


def kernel(x, Wg, W1, W2):
    Bb, Tt, C = x.shape
    x_flat = x.reshape(N, D)
    slot0, slot1, sorted_tok, w_sorted, block_expert = _routing(x_flat, Wg)
    x_sorted = _sc_gather(sorted_tok.reshape(S), x_flat)
    out_sorted = _ffn(block_expert, x_sorted, W1, W2, w_sorted)
    out = _sc_combine(slot0, slot1, out_sorted)
    return out.reshape(Bb, Tt, C)


# speedup vs baseline: 1.2038x; 1.2038x over previous
import functools

import jax
import jax.numpy as jnp
from jax import lax
from jax.experimental import pallas as pl
from jax.experimental.pallas import tpu as pltpu
from jax.experimental.pallas import tpu_sc as plsc

E = 8
TOP_K = 2
D = 1024
H = 4096
N = 2048
BLK = 128
G = (TOP_K * N) // BLK + E
S = G * BLK
HT = 512
NH = H // HT

NEG_BIG = -1e30


def _routing_body(x_ref, wg_ref, slot0_ref, slot1_ref, tok_ref, ws_ref, be_ref):
    x = x_ref[...]
    wg = wg_ref[...]
    logits = lax.dot_general(x, wg, (((1,), (1,)), ((), ())),
                             preferred_element_type=jnp.float32)
    m = jnp.max(logits, axis=-1, keepdims=True)
    ex = jnp.exp(logits - m)
    gw = ex / jnp.sum(ex, axis=-1, keepdims=True)

    iota8 = lax.broadcasted_iota(jnp.int32, (N, E), 1)
    m0 = jnp.max(gw, axis=-1, keepdims=True)
    e0 = jnp.min(jnp.where(gw == m0, iota8, E), axis=-1)
    gw1 = jnp.where(iota8 == e0[:, None], NEG_BIG, gw)
    m1 = jnp.max(gw1, axis=-1, keepdims=True)
    e1 = jnp.min(jnp.where(gw1 == m1, iota8, E), axis=-1)
    w0 = m0[:, 0]
    w1 = m1[:, 0]
    denom = w0 + w1 + 1e-9
    w0 = w0 / denom
    w1 = w1 / denom

    oh0 = (iota8 == e0[:, None]).astype(jnp.float32)
    oh1 = (iota8 == e1[:, None]).astype(jnp.float32)

    ltri = (lax.broadcasted_iota(jnp.int32, (BLK, BLK), 0)
            > lax.broadcasted_iota(jnp.int32, (BLK, BLK), 1)).astype(jnp.float32)
    nchunks = N // BLK

    def excl_cumsum(oh, start):
        parts = []
        carry = start
        for c in range(nchunks):
            ch = oh[c * BLK:(c + 1) * BLK, :]
            parts.append(carry + lax.dot_general(
                ltri, ch, (((1,), (0,)), ((), ())),
                preferred_element_type=jnp.float32))
            carry = carry + jnp.sum(ch, axis=0, keepdims=True)
        return jnp.concatenate(parts, axis=0), carry

    zero_row = jnp.zeros((1, E), jnp.float32)
    rank0, c0 = excl_cumsum(oh0, zero_row)
    rank1, ctot = excl_cumsum(oh1, c0)

    counts = ctot[0]
    nblk = jnp.floor((counts + (BLK - 1)) / BLK)
    ltri_e = (lax.broadcasted_iota(jnp.int32, (E, E), 0)
              > lax.broadcasted_iota(jnp.int32, (E, E), 1)).astype(jnp.float32)
    padded_off = BLK * jnp.dot(ltri_e, nblk,
                               preferred_element_type=jnp.float32)

    slot0f = jnp.sum((padded_off[None, :] + rank0) * oh0, axis=-1)
    slot1f = jnp.sum((padded_off[None, :] + rank1) * oh1, axis=-1)
    slot0_ref[...] = slot0f.astype(jnp.int32)
    slot1_ref[...] = slot1f.astype(jnp.int32)

    gidx = (lax.broadcasted_iota(jnp.int32, (G, E - 1), 0) * BLK).astype(jnp.float32)
    starts = padded_off[None, 1:]
    be_ref[...] = jnp.sum((gidx >= starts).astype(jnp.int32), axis=-1)

    tokf = lax.broadcasted_iota(jnp.int32, (N, 1), 0).astype(jnp.float32)
    s0 = slot0f[None, :]
    s1 = slot1f[None, :]
    rows0 = lax.broadcasted_iota(jnp.int32, (BLK, 1), 0).astype(jnp.float32)
    for b in range(G):
        rows = rows0 + b * BLK
        m0b = (rows == s0).astype(jnp.float32)
        m1b = (rows == s1).astype(jnp.float32)
        tok_b = (jnp.dot(m0b, tokf, preferred_element_type=jnp.float32)
                 + jnp.dot(m1b, tokf, preferred_element_type=jnp.float32))
        ws_b = (jnp.dot(m0b, w0[:, None], preferred_element_type=jnp.float32)
                + jnp.dot(m1b, w1[:, None], preferred_element_type=jnp.float32))
        tok_ref[b * BLK:(b + 1) * BLK, :] = tok_b.astype(jnp.int32)
        ws_ref[b * BLK:(b + 1) * BLK, :] = ws_b


def _routing(x_flat, Wg):
    return pl.pallas_call(
        _routing_body,
        out_shape=[
            jax.ShapeDtypeStruct((N,), jnp.int32),
            jax.ShapeDtypeStruct((N,), jnp.int32),
            jax.ShapeDtypeStruct((S, 1), jnp.int32),
            jax.ShapeDtypeStruct((S, 1), jnp.float32),
            jax.ShapeDtypeStruct((G,), jnp.int32),
        ],
    )(x_flat, Wg)


_NC = 2
_NS = 16
_NW = _NC * _NS
_GCH = 32


def _sc_gather_body(tok_hbm, x_hbm, out_hbm, idx_v, rows_v, sem):
    wid = lax.axis_index("s") * _NC + lax.axis_index("c")
    per_w = S // _NW
    base = wid * per_w
    for c in range(per_w // _GCH):
        off = base + c * _GCH
        pltpu.sync_copy(tok_hbm.at[pl.ds(off, _GCH)], idx_v)
        pltpu.async_copy(x_hbm.at[idx_v], rows_v, sem).wait()
        pltpu.sync_copy(rows_v, out_hbm.at[pl.ds(off, _GCH)])


def _sc_gather(sorted_tok, x_flat):
    mesh = plsc.VectorSubcoreMesh(core_axis_name="c", subcore_axis_name="s")
    fn = functools.partial(
        pl.kernel, mesh=mesh,
        out_type=jax.ShapeDtypeStruct((S, D), jnp.float32),
        scratch_types=[
            pltpu.VMEM((_GCH,), jnp.int32),
            pltpu.VMEM((_GCH, D), jnp.float32),
            pltpu.SemaphoreType.DMA,
        ],
    )(_sc_gather_body)
    return fn(sorted_tok, x_flat)


def _ffn_body(be_ref, x_ref, w1_ref, w2_ref, ws_ref, out_ref):
    h = pl.program_id(0)
    g = pl.program_id(1)
    xb = x_ref[pl.ds(g * BLK, BLK), :]
    h1 = lax.dot_general(xb, w1_ref[0], (((1,), (1,)), ((), ())),
                         preferred_element_type=jnp.float32)
    h1 = jnp.maximum(h1, 0.0)
    contrib = lax.dot_general(h1, w2_ref[0], (((1,), (1,)), ((), ())),
                              preferred_element_type=jnp.float32)

    @pl.when(h == 0)
    def _():
        out_ref[pl.ds(g * BLK, BLK), :] = contrib

    @pl.when((h > 0) & (h < NH - 1))
    def _():
        out_ref[pl.ds(g * BLK, BLK), :] = out_ref[pl.ds(g * BLK, BLK), :] + contrib

    @pl.when(h == NH - 1)
    def _():
        out_ref[pl.ds(g * BLK, BLK), :] = (
            (out_ref[pl.ds(g * BLK, BLK), :] + contrib)
            * ws_ref[pl.ds(g * BLK, BLK), :])


def _ffn(block_expert, x_sorted, W1, W2, w_sorted):
    grid_spec = pltpu.PrefetchScalarGridSpec(
        num_scalar_prefetch=1,
        grid=(NH, G),
        in_specs=[
            pl.BlockSpec((S, D), lambda h, g, be: (0, 0)),
            pl.BlockSpec((1, HT, D), lambda h, g, be: (be[g], h, 0)),
            pl.BlockSpec((1, D, HT), lambda h, g, be: (be[g], 0, h)),
            pl.BlockSpec((S, 1), lambda h, g, be: (0, 0)),
        ],
        out_specs=pl.BlockSpec((S, D), lambda h, g, be: (0, 0)),
    )
    return pl.pallas_call(
        _ffn_body,
        grid_spec=grid_spec,
        out_shape=jax.ShapeDtypeStruct((S, D), jnp.float32),
    )(block_expert, x_sorted, W1, W2, w_sorted)


_CCH = 32


def _sc_combine_body(s0_hbm, s1_hbm, os_hbm, out_hbm, idx_v, a_v, b_v, sem):
    wid = lax.axis_index("s") * _NC + lax.axis_index("c")
    per_w = N // _NW
    base = wid * per_w
    for c in range(per_w // _CCH):
        off = base + c * _CCH
        pltpu.sync_copy(s0_hbm.at[pl.ds(off, _CCH)], idx_v)
        pltpu.async_copy(os_hbm.at[idx_v], a_v, sem).wait()
        pltpu.sync_copy(s1_hbm.at[pl.ds(off, _CCH)], idx_v)
        pltpu.async_copy(os_hbm.at[idx_v], b_v, sem).wait()

        def add_row(r, carry):
            for j in range(D // 16):
                a_v[r, pl.ds(j * 16, 16)] = (a_v[r, pl.ds(j * 16, 16)]
                                             + b_v[r, pl.ds(j * 16, 16)])
            return carry

        lax.fori_loop(0, _CCH, add_row, 0)
        pltpu.sync_copy(a_v, out_hbm.at[pl.ds(off, _CCH)])


def _sc_combine(slot0, slot1, out_sorted):
    mesh = plsc.VectorSubcoreMesh(core_axis_name="c", subcore_axis_name="s")
    fn = functools.partial(
        pl.kernel, mesh=mesh,
        out_type=jax.ShapeDtypeStruct((N, D), jnp.float32),
        scratch_types=[
            pltpu.VMEM((_CCH,), jnp.int32),
            pltpu.VMEM((_CCH, D), jnp.float32),
            pltpu.VMEM((_CCH, D), jnp.float32),
            pltpu.SemaphoreType.DMA,
        ],
    )(_sc_combine_body)
    return fn(slot0, slot1, out_sorted)


def kernel(x, Wg, W1, W2):
    Bb, Tt, C = x.shape
    x_flat = x.reshape(N, D)
    slot0, slot1, sorted_tok, w_sorted, block_expert = _routing(x_flat, Wg)
    x_sorted = _sc_gather(sorted_tok.reshape(S), x_flat)
    out_sorted = _ffn(block_expert, x_sorted, W1, W2, w_sorted)
    out = _sc_combine(slot0, slot1, out_sorted)
    return out.reshape(Bb, Tt, C)


# speedup vs baseline: 1.2373x; 1.0279x over previous
import functools

import jax
import jax.numpy as jnp
from jax import lax
from jax.experimental import pallas as pl
from jax.experimental.pallas import tpu as pltpu
from jax.experimental.pallas import tpu_sc as plsc

E = 8
TOP_K = 2
D = 1024
H = 4096
N = 2048
BLK = 128
G = (TOP_K * N) // BLK + E
S = G * BLK
HT = 512
NH = H // HT

NEG_BIG = -1e30


def _routing_body(x_ref, wg_ref, slot0_ref, slot1_ref, w0_ref, w1_ref, be_ref):
    x = x_ref[...]
    wg = wg_ref[...]
    logits = lax.dot_general(x, wg, (((1,), (1,)), ((), ())),
                             preferred_element_type=jnp.float32)
    m = jnp.max(logits, axis=-1, keepdims=True)
    ex = jnp.exp(logits - m)
    gw = ex / jnp.sum(ex, axis=-1, keepdims=True)

    iota8 = lax.broadcasted_iota(jnp.int32, (N, E), 1)
    m0 = jnp.max(gw, axis=-1, keepdims=True)
    e0 = jnp.min(jnp.where(gw == m0, iota8, E), axis=-1)
    gw1 = jnp.where(iota8 == e0[:, None], NEG_BIG, gw)
    m1 = jnp.max(gw1, axis=-1, keepdims=True)
    e1 = jnp.min(jnp.where(gw1 == m1, iota8, E), axis=-1)
    w0 = m0[:, 0]
    w1 = m1[:, 0]
    denom = w0 + w1 + 1e-9
    w0 = w0 / denom
    w1 = w1 / denom
    ones16 = jnp.ones((1, 16), jnp.float32)
    w0_ref[...] = w0[:, None] * ones16
    w1_ref[...] = w1[:, None] * ones16

    oh0 = (iota8 == e0[:, None]).astype(jnp.float32)
    oh1 = (iota8 == e1[:, None]).astype(jnp.float32)

    ltri = (lax.broadcasted_iota(jnp.int32, (BLK, BLK), 0)
            > lax.broadcasted_iota(jnp.int32, (BLK, BLK), 1)).astype(jnp.float32)
    nchunks = N // BLK

    def excl_cumsum(oh, start):
        parts = []
        carry = start
        for c in range(nchunks):
            ch = oh[c * BLK:(c + 1) * BLK, :]
            parts.append(carry + lax.dot_general(
                ltri, ch, (((1,), (0,)), ((), ())),
                preferred_element_type=jnp.float32))
            carry = carry + jnp.sum(ch, axis=0, keepdims=True)
        return jnp.concatenate(parts, axis=0), carry

    zero_row = jnp.zeros((1, E), jnp.float32)
    rank0, c0 = excl_cumsum(oh0, zero_row)
    rank1, ctot = excl_cumsum(oh1, c0)

    counts = ctot[0]
    nblk = jnp.floor((counts + (BLK - 1)) / BLK)
    ltri_e = (lax.broadcasted_iota(jnp.int32, (E, E), 0)
              > lax.broadcasted_iota(jnp.int32, (E, E), 1)).astype(jnp.float32)
    padded_off = BLK * jnp.dot(ltri_e, nblk,
                               preferred_element_type=jnp.float32)

    slot0f = jnp.sum((padded_off[None, :] + rank0) * oh0, axis=-1)
    slot1f = jnp.sum((padded_off[None, :] + rank1) * oh1, axis=-1)
    slot0_ref[...] = slot0f.astype(jnp.int32)
    slot1_ref[...] = slot1f.astype(jnp.int32)

    gidx = (lax.broadcasted_iota(jnp.int32, (G, E - 1), 0) * BLK).astype(jnp.float32)
    starts = padded_off[None, 1:]
    be_ref[...] = jnp.sum((gidx >= starts).astype(jnp.int32), axis=-1)


def _routing(x_flat, Wg):
    return pl.pallas_call(
        _routing_body,
        out_shape=[
            jax.ShapeDtypeStruct((N,), jnp.int32),
            jax.ShapeDtypeStruct((N,), jnp.int32),
            jax.ShapeDtypeStruct((N, 16), jnp.float32),
            jax.ShapeDtypeStruct((N, 16), jnp.float32),
            jax.ShapeDtypeStruct((G,), jnp.int32),
        ],
    )(x_flat, Wg)


_NC = 2
_NS = 16
_NW = _NC * _NS
_SCH = 32


def _sc_scatter_body(s0_hbm, s1_hbm, x_hbm, out_hbm, idx0_v, idx1_v, rows_v, sem):
    wid = lax.axis_index("s") * _NC + lax.axis_index("c")
    per_w = N // _NW
    base = wid * per_w
    for c in range(per_w // _SCH):
        off = base + c * _SCH
        pltpu.sync_copy(s0_hbm.at[pl.ds(off, _SCH)], idx0_v.at[0])
        pltpu.sync_copy(s1_hbm.at[pl.ds(off, _SCH)], idx1_v.at[0])
        pltpu.sync_copy(x_hbm.at[pl.ds(off, _SCH)], rows_v)
        cp0 = pltpu.async_copy(rows_v, out_hbm.at[idx0_v.at[0]], sem)
        cp1 = pltpu.async_copy(rows_v, out_hbm.at[idx1_v.at[0]], sem)
        cp0.wait()
        cp1.wait()


def _sc_scatter(slot0, slot1, x_flat):
    mesh = plsc.VectorSubcoreMesh(core_axis_name="c", subcore_axis_name="s")
    fn = functools.partial(
        pl.kernel, mesh=mesh,
        out_type=jax.ShapeDtypeStruct((S, D), jnp.float32),
        scratch_types=[
            pltpu.VMEM((1, _SCH), jnp.int32),
            pltpu.VMEM((1, _SCH), jnp.int32),
            pltpu.VMEM((_SCH, D), jnp.float32),
            pltpu.SemaphoreType.DMA,
        ],
    )(_sc_scatter_body)
    return fn(slot0, slot1, x_flat)


def _ffn_body(be_ref, x_ref, w1_ref, w2_ref, out_ref):
    h = pl.program_id(0)
    g = pl.program_id(1)
    xb = x_ref[pl.ds(g * BLK, BLK), :].astype(jnp.bfloat16)
    h1 = lax.dot_general(xb, w1_ref[0], (((1,), (1,)), ((), ())),
                         preferred_element_type=jnp.float32)
    h1 = jnp.maximum(h1, 0.0).astype(jnp.bfloat16)
    contrib = lax.dot_general(h1, w2_ref[0], (((1,), (1,)), ((), ())),
                              preferred_element_type=jnp.float32)

    @pl.when(h == 0)
    def _():
        out_ref[pl.ds(g * BLK, BLK), :] = contrib

    @pl.when(h > 0)
    def _():
        out_ref[pl.ds(g * BLK, BLK), :] = out_ref[pl.ds(g * BLK, BLK), :] + contrib


def _ffn(block_expert, x_sorted, W1, W2):
    grid_spec = pltpu.PrefetchScalarGridSpec(
        num_scalar_prefetch=1,
        grid=(NH, G),
        in_specs=[
            pl.BlockSpec((S, D), lambda h, g, be: (0, 0)),
            pl.BlockSpec((1, HT, D), lambda h, g, be: (be[g], h, 0)),
            pl.BlockSpec((1, D, HT), lambda h, g, be: (be[g], 0, h)),
        ],
        out_specs=pl.BlockSpec((S, D), lambda h, g, be: (0, 0)),
    )
    return pl.pallas_call(
        _ffn_body,
        grid_spec=grid_spec,
        out_shape=jax.ShapeDtypeStruct((S, D), jnp.float32),
    )(block_expert, x_sorted, W1, W2)


_CCH = 32


def _sc_combine_body(s0_hbm, s1_hbm, w0_hbm, w1_hbm, os_hbm, out_hbm,
                     idx_v, a_v, b_v, w0_v, w1_v, sem):
    wid = lax.axis_index("s") * _NC + lax.axis_index("c")
    per_w = N // _NW
    base = wid * per_w
    for c in range(per_w // _CCH):
        off = base + c * _CCH
        pltpu.sync_copy(s0_hbm.at[pl.ds(off, _CCH)], idx_v)
        pltpu.async_copy(os_hbm.at[idx_v], a_v, sem).wait()
        pltpu.sync_copy(s1_hbm.at[pl.ds(off, _CCH)], idx_v)
        pltpu.async_copy(os_hbm.at[idx_v], b_v, sem).wait()
        pltpu.sync_copy(w0_hbm.at[pl.ds(off, _CCH)], w0_v)
        pltpu.sync_copy(w1_hbm.at[pl.ds(off, _CCH)], w1_v)

        def combine_row(r, carry):
            wa = w0_v[r, :]
            wb = w1_v[r, :]
            for j in range(D // 16):
                a_v[r, pl.ds(j * 16, 16)] = (wa * a_v[r, pl.ds(j * 16, 16)]
                                             + wb * b_v[r, pl.ds(j * 16, 16)])
            return carry

        lax.fori_loop(0, _CCH, combine_row, 0)
        pltpu.sync_copy(a_v, out_hbm.at[pl.ds(off, _CCH)])


def _sc_combine(slot0, slot1, w0r, w1r, out_sorted):
    mesh = plsc.VectorSubcoreMesh(core_axis_name="c", subcore_axis_name="s")
    fn = functools.partial(
        pl.kernel, mesh=mesh,
        out_type=jax.ShapeDtypeStruct((N, D), jnp.float32),
        scratch_types=[
            pltpu.VMEM((_CCH,), jnp.int32),
            pltpu.VMEM((_CCH, D), jnp.float32),
            pltpu.VMEM((_CCH, D), jnp.float32),
            pltpu.VMEM((_CCH, 16), jnp.float32),
            pltpu.VMEM((_CCH, 16), jnp.float32),
            pltpu.SemaphoreType.DMA,
        ],
    )(_sc_combine_body)
    return fn(slot0, slot1, w0r, w1r, out_sorted)


def kernel(x, Wg, W1, W2):
    Bb, Tt, C = x.shape
    x_flat = x.reshape(N, D)
    slot0, slot1, w0r, w1r, block_expert = _routing(x_flat, Wg)
    x_sorted = _sc_scatter(slot0, slot1, x_flat)
    out_sorted = _ffn(block_expert, x_sorted,
                      W1.astype(jnp.bfloat16), W2.astype(jnp.bfloat16))
    out = _sc_combine(slot0, slot1, w0r, w1r, out_sorted)
    return out.reshape(Bb, Tt, C)


# speedup vs baseline: 1.5843x; 1.2804x over previous
import functools

import jax
import jax.numpy as jnp
from jax import lax
from jax.experimental import pallas as pl
from jax.experimental.pallas import tpu as pltpu
from jax.experimental.pallas import tpu_sc as plsc

E = 8
TOP_K = 2
D = 1024
H = 4096
N = 2048
BLK = 256
G = (TOP_K * N) // BLK + E
S = G * BLK
HT = 512
NH = H // HT

NEG_BIG = -1e30


def _routing_body(x_ref, wg_ref, slot0_ref, slot1_ref, w0_ref, w1_ref, be_ref):
    x = x_ref[...]
    wg = wg_ref[...]
    logits = lax.dot_general(x, wg, (((1,), (1,)), ((), ())),
                             preferred_element_type=jnp.float32)
    m = jnp.max(logits, axis=-1, keepdims=True)
    ex = jnp.exp(logits - m)
    gw = ex / jnp.sum(ex, axis=-1, keepdims=True)

    iota8 = lax.broadcasted_iota(jnp.int32, (N, E), 1)
    m0 = jnp.max(gw, axis=-1, keepdims=True)
    e0 = jnp.min(jnp.where(gw == m0, iota8, E), axis=-1)
    gw1 = jnp.where(iota8 == e0[:, None], NEG_BIG, gw)
    m1 = jnp.max(gw1, axis=-1, keepdims=True)
    e1 = jnp.min(jnp.where(gw1 == m1, iota8, E), axis=-1)
    w0 = m0[:, 0]
    w1 = m1[:, 0]
    denom = w0 + w1 + 1e-9
    w0 = w0 / denom
    w1 = w1 / denom
    ones16 = jnp.ones((1, 16), jnp.float32)
    w0_ref[...] = w0[:, None] * ones16
    w1_ref[...] = w1[:, None] * ones16

    oh0 = (iota8 == e0[:, None]).astype(jnp.float32)
    oh1 = (iota8 == e1[:, None]).astype(jnp.float32)

    ltri = (lax.broadcasted_iota(jnp.int32, (BLK, BLK), 0)
            > lax.broadcasted_iota(jnp.int32, (BLK, BLK), 1)).astype(jnp.float32)
    nchunks = N // BLK

    def excl_cumsum(oh, start):
        parts = []
        carry = start
        for c in range(nchunks):
            ch = oh[c * BLK:(c + 1) * BLK, :]
            parts.append(carry + lax.dot_general(
                ltri, ch, (((1,), (0,)), ((), ())),
                preferred_element_type=jnp.float32))
            carry = carry + jnp.sum(ch, axis=0, keepdims=True)
        return jnp.concatenate(parts, axis=0), carry

    zero_row = jnp.zeros((1, E), jnp.float32)
    rank0, c0 = excl_cumsum(oh0, zero_row)
    rank1, ctot = excl_cumsum(oh1, c0)

    counts = ctot[0]
    nblk = jnp.floor((counts + (BLK - 1)) / BLK)
    ltri_e = (lax.broadcasted_iota(jnp.int32, (E, E), 0)
              > lax.broadcasted_iota(jnp.int32, (E, E), 1)).astype(jnp.float32)
    padded_off = BLK * jnp.dot(ltri_e, nblk,
                               preferred_element_type=jnp.float32)

    slot0f = jnp.sum((padded_off[None, :] + rank0) * oh0, axis=-1)
    slot1f = jnp.sum((padded_off[None, :] + rank1) * oh1, axis=-1)
    slot0_ref[...] = slot0f.astype(jnp.int32)
    slot1_ref[...] = slot1f.astype(jnp.int32)

    gidx = (lax.broadcasted_iota(jnp.int32, (G, E - 1), 0) * BLK).astype(jnp.float32)
    starts = padded_off[None, 1:]
    be_ref[...] = jnp.sum((gidx >= starts).astype(jnp.int32), axis=-1)


def _routing(x_flat, Wg):
    return pl.pallas_call(
        _routing_body,
        out_shape=[
            jax.ShapeDtypeStruct((N,), jnp.int32),
            jax.ShapeDtypeStruct((N,), jnp.int32),
            jax.ShapeDtypeStruct((N, 16), jnp.float32),
            jax.ShapeDtypeStruct((N, 16), jnp.float32),
            jax.ShapeDtypeStruct((G,), jnp.int32),
        ],
    )(x_flat, Wg)


_NC = 2
_NS = 16
_NW = _NC * _NS
_SCH = 32


def _sc_scatter_body(s0_hbm, s1_hbm, x_hbm, out_hbm, idx0_v, idx1_v, rows_v, sem):
    wid = lax.axis_index("s") * _NC + lax.axis_index("c")
    per_w = N // _NW
    base = wid * per_w
    for c in range(per_w // _SCH):
        off = base + c * _SCH
        pltpu.sync_copy(s0_hbm.at[pl.ds(off, _SCH)], idx0_v.at[0])
        pltpu.sync_copy(s1_hbm.at[pl.ds(off, _SCH)], idx1_v.at[0])
        pltpu.sync_copy(x_hbm.at[pl.ds(off, _SCH)], rows_v)
        cp0 = pltpu.async_copy(rows_v, out_hbm.at[idx0_v.at[0]], sem)
        cp1 = pltpu.async_copy(rows_v, out_hbm.at[idx1_v.at[0]], sem)
        cp0.wait()
        cp1.wait()


def _sc_scatter(slot0, slot1, x_flat):
    mesh = plsc.VectorSubcoreMesh(core_axis_name="c", subcore_axis_name="s")
    fn = functools.partial(
        pl.kernel, mesh=mesh,
        out_type=jax.ShapeDtypeStruct((S, D), jnp.float32),
        scratch_types=[
            pltpu.VMEM((1, _SCH), jnp.int32),
            pltpu.VMEM((1, _SCH), jnp.int32),
            pltpu.VMEM((_SCH, D), jnp.float32),
            pltpu.SemaphoreType.DMA,
        ],
    )(_sc_scatter_body)
    return fn(slot0, slot1, x_flat)


def _ffn_body(be_ref, x_ref, w1_ref, w2_ref, out_ref):
    h = pl.program_id(0)
    g = pl.program_id(1)
    xb = x_ref[pl.ds(g * BLK, BLK), :].astype(jnp.bfloat16)
    h1 = lax.dot_general(xb, w1_ref[0], (((1,), (1,)), ((), ())),
                         preferred_element_type=jnp.float32)
    h1 = jnp.maximum(h1, 0.0).astype(jnp.bfloat16)
    contrib = lax.dot_general(h1, w2_ref[0], (((1,), (1,)), ((), ())),
                              preferred_element_type=jnp.float32)

    @pl.when(h == 0)
    def _():
        out_ref[pl.ds(g * BLK, BLK), :] = contrib

    @pl.when(h > 0)
    def _():
        out_ref[pl.ds(g * BLK, BLK), :] = out_ref[pl.ds(g * BLK, BLK), :] + contrib


def _ffn(block_expert, x_sorted, W1, W2):
    grid_spec = pltpu.PrefetchScalarGridSpec(
        num_scalar_prefetch=1,
        grid=(NH, G),
        in_specs=[
            pl.BlockSpec((S, D), lambda h, g, be: (0, 0)),
            pl.BlockSpec((1, HT, D), lambda h, g, be: (be[g], h, 0)),
            pl.BlockSpec((1, D, HT), lambda h, g, be: (be[g], 0, h)),
        ],
        out_specs=pl.BlockSpec((S, D), lambda h, g, be: (0, 0)),
    )
    return pl.pallas_call(
        _ffn_body,
        grid_spec=grid_spec,
        out_shape=jax.ShapeDtypeStruct((S, D), jnp.float32),
    )(block_expert, x_sorted, W1, W2)


_CCH = 32


def _sc_combine_body(s0_hbm, s1_hbm, w0_hbm, w1_hbm, os_hbm, out_hbm,
                     idx_v, a_v, b_v, w0_v, w1_v, sem):
    wid = lax.axis_index("s") * _NC + lax.axis_index("c")
    per_w = N // _NW
    base = wid * per_w
    for c in range(per_w // _CCH):
        off = base + c * _CCH
        pltpu.sync_copy(s0_hbm.at[pl.ds(off, _CCH)], idx_v)
        pltpu.async_copy(os_hbm.at[idx_v], a_v, sem).wait()
        pltpu.sync_copy(s1_hbm.at[pl.ds(off, _CCH)], idx_v)
        pltpu.async_copy(os_hbm.at[idx_v], b_v, sem).wait()
        pltpu.sync_copy(w0_hbm.at[pl.ds(off, _CCH)], w0_v)
        pltpu.sync_copy(w1_hbm.at[pl.ds(off, _CCH)], w1_v)

        def combine_row(r, carry):
            wa = w0_v[r, :]
            wb = w1_v[r, :]
            for j in range(D // 16):
                a_v[r, pl.ds(j * 16, 16)] = (wa * a_v[r, pl.ds(j * 16, 16)]
                                             + wb * b_v[r, pl.ds(j * 16, 16)])
            return carry

        lax.fori_loop(0, _CCH, combine_row, 0)
        pltpu.sync_copy(a_v, out_hbm.at[pl.ds(off, _CCH)])


def _sc_combine(slot0, slot1, w0r, w1r, out_sorted):
    mesh = plsc.VectorSubcoreMesh(core_axis_name="c", subcore_axis_name="s")
    fn = functools.partial(
        pl.kernel, mesh=mesh,
        out_type=jax.ShapeDtypeStruct((N, D), jnp.float32),
        scratch_types=[
            pltpu.VMEM((_CCH,), jnp.int32),
            pltpu.VMEM((_CCH, D), jnp.float32),
            pltpu.VMEM((_CCH, D), jnp.float32),
            pltpu.VMEM((_CCH, 16), jnp.float32),
            pltpu.VMEM((_CCH, 16), jnp.float32),
            pltpu.SemaphoreType.DMA,
        ],
    )(_sc_combine_body)
    return fn(slot0, slot1, w0r, w1r, out_sorted)


def kernel(x, Wg, W1, W2):
    Bb, Tt, C = x.shape
    x_flat = x.reshape(N, D)
    slot0, slot1, w0r, w1r, block_expert = _routing(x_flat, Wg)
    x_sorted = _sc_scatter(slot0, slot1, x_flat)
    out_sorted = _ffn(block_expert, x_sorted,
                      W1.astype(jnp.bfloat16), W2.astype(jnp.bfloat16))
    out = _sc_combine(slot0, slot1, w0r, w1r, out_sorted)
    return out.reshape(Bb, Tt, C)


# speedup vs baseline: 1.9155x; 1.2091x over previous
import functools

import jax
import jax.numpy as jnp
from jax import lax
from jax.experimental import pallas as pl
from jax.experimental.pallas import tpu as pltpu
from jax.experimental.pallas import tpu_sc as plsc

E = 8
TOP_K = 2
D = 1024
H = 4096
N = 2048
BLK = 256
G = (TOP_K * N) // BLK + E
S = G * BLK
HT = 512
NH = H // HT

NEG_BIG = -1e30


def _routing_body(x_ref, wg_ref, slot0_ref, slot1_ref, w0_ref, w1_ref, be_ref):
    x = x_ref[...]
    wg = wg_ref[...]
    logits = lax.dot_general(x, wg, (((1,), (1,)), ((), ())),
                             preferred_element_type=jnp.float32)
    m = jnp.max(logits, axis=-1, keepdims=True)
    ex = jnp.exp(logits - m)
    gw = ex / jnp.sum(ex, axis=-1, keepdims=True)

    iota8 = lax.broadcasted_iota(jnp.int32, (N, E), 1)
    m0 = jnp.max(gw, axis=-1, keepdims=True)
    e0 = jnp.min(jnp.where(gw == m0, iota8, E), axis=-1)
    gw1 = jnp.where(iota8 == e0[:, None], NEG_BIG, gw)
    m1 = jnp.max(gw1, axis=-1, keepdims=True)
    e1 = jnp.min(jnp.where(gw1 == m1, iota8, E), axis=-1)
    w0 = m0[:, 0]
    w1 = m1[:, 0]
    denom = w0 + w1 + 1e-9
    w0 = w0 / denom
    w1 = w1 / denom
    ones16 = jnp.ones((1, 16), jnp.float32)
    w0_ref[...] = w0[:, None] * ones16
    w1_ref[...] = w1[:, None] * ones16

    oh0 = (iota8 == e0[:, None]).astype(jnp.float32)
    oh1 = (iota8 == e1[:, None]).astype(jnp.float32)

    ltri = (lax.broadcasted_iota(jnp.int32, (BLK, BLK), 0)
            > lax.broadcasted_iota(jnp.int32, (BLK, BLK), 1)).astype(jnp.float32)
    nchunks = N // BLK

    def excl_cumsum(oh, start):
        parts = []
        carry = start
        for c in range(nchunks):
            ch = oh[c * BLK:(c + 1) * BLK, :]
            parts.append(carry + lax.dot_general(
                ltri, ch, (((1,), (0,)), ((), ())),
                preferred_element_type=jnp.float32))
            carry = carry + jnp.sum(ch, axis=0, keepdims=True)
        return jnp.concatenate(parts, axis=0), carry

    zero_row = jnp.zeros((1, E), jnp.float32)
    rank0, c0 = excl_cumsum(oh0, zero_row)
    rank1, ctot = excl_cumsum(oh1, c0)

    counts = ctot[0]
    nblk = jnp.floor((counts + (BLK - 1)) / BLK)
    ltri_e = (lax.broadcasted_iota(jnp.int32, (E, E), 0)
              > lax.broadcasted_iota(jnp.int32, (E, E), 1)).astype(jnp.float32)
    padded_off = BLK * jnp.dot(ltri_e, nblk,
                               preferred_element_type=jnp.float32)

    slot0f = jnp.sum((padded_off[None, :] + rank0) * oh0, axis=-1)
    slot1f = jnp.sum((padded_off[None, :] + rank1) * oh1, axis=-1)
    slot0_ref[...] = slot0f.astype(jnp.int32)
    slot1_ref[...] = slot1f.astype(jnp.int32)

    gidx = (lax.broadcasted_iota(jnp.int32, (G, E - 1), 0) * BLK).astype(jnp.float32)
    starts = padded_off[None, 1:]
    be_ref[...] = jnp.sum((gidx >= starts).astype(jnp.int32), axis=-1)


def _routing(x_flat, Wg):
    return pl.pallas_call(
        _routing_body,
        out_shape=[
            jax.ShapeDtypeStruct((N,), jnp.int32),
            jax.ShapeDtypeStruct((N,), jnp.int32),
            jax.ShapeDtypeStruct((N, 16), jnp.float32),
            jax.ShapeDtypeStruct((N, 16), jnp.float32),
            jax.ShapeDtypeStruct((G,), jnp.int32),
        ],
    )(x_flat, Wg)


_NC = 2
_NS = 16
_NW = _NC * _NS
_SCH = 32


def _sc_scatter_body(s0_hbm, s1_hbm, x_hbm, out_hbm, idx0_v, idx1_v, rows_v, sem):
    wid = lax.axis_index("s") * _NC + lax.axis_index("c")
    per_w = N // _NW
    base = wid * per_w
    for c in range(per_w // _SCH):
        off = base + c * _SCH
        pltpu.sync_copy(s0_hbm.at[pl.ds(off, _SCH)], idx0_v.at[0])
        pltpu.sync_copy(s1_hbm.at[pl.ds(off, _SCH)], idx1_v.at[0])
        pltpu.sync_copy(x_hbm.at[pl.ds(off, _SCH)], rows_v)
        cp0 = pltpu.async_copy(rows_v, out_hbm.at[idx0_v.at[0]], sem)
        cp1 = pltpu.async_copy(rows_v, out_hbm.at[idx1_v.at[0]], sem)
        cp0.wait()
        cp1.wait()


def _sc_scatter(slot0, slot1, x_flat):
    mesh = plsc.VectorSubcoreMesh(core_axis_name="c", subcore_axis_name="s")
    fn = functools.partial(
        pl.kernel, mesh=mesh,
        out_type=jax.ShapeDtypeStruct((S, D), jnp.float32),
        scratch_types=[
            pltpu.VMEM((1, _SCH), jnp.int32),
            pltpu.VMEM((1, _SCH), jnp.int32),
            pltpu.VMEM((_SCH, D), jnp.float32),
            pltpu.SemaphoreType.DMA,
        ],
    )(_sc_scatter_body)
    return fn(slot0, slot1, x_flat)


def _ffn_body(be_ref, x_ref, w1_ref, w2_ref, out_ref):
    h = pl.program_id(0)
    g = pl.program_id(1)
    xb = x_ref[pl.ds(g * BLK, BLK), :]
    h1 = lax.dot_general(xb, w1_ref[0], (((1,), (1,)), ((), ())),
                         preferred_element_type=jnp.float32)
    h1 = jnp.maximum(h1, 0.0)
    contrib = lax.dot_general(h1, w2_ref[0], (((1,), (1,)), ((), ())),
                              preferred_element_type=jnp.float32)

    @pl.when(h == 0)
    def _():
        out_ref[pl.ds(g * BLK, BLK), :] = contrib

    @pl.when(h > 0)
    def _():
        out_ref[pl.ds(g * BLK, BLK), :] = out_ref[pl.ds(g * BLK, BLK), :] + contrib


def _ffn(block_expert, x_sorted, W1, W2):
    grid_spec = pltpu.PrefetchScalarGridSpec(
        num_scalar_prefetch=1,
        grid=(NH, G),
        in_specs=[
            pl.BlockSpec((S, D), lambda h, g, be: (0, 0)),
            pl.BlockSpec((1, HT, D), lambda h, g, be: (be[g], h, 0)),
            pl.BlockSpec((1, D, HT), lambda h, g, be: (be[g], 0, h)),
        ],
        out_specs=pl.BlockSpec((S, D), lambda h, g, be: (0, 0)),
    )
    return pl.pallas_call(
        _ffn_body,
        grid_spec=grid_spec,
        out_shape=jax.ShapeDtypeStruct((S, D), jnp.float32),
    )(block_expert, x_sorted, W1, W2)


_CCH = 32


def _sc_combine_body(s0_hbm, s1_hbm, w0_hbm, w1_hbm, os_hbm, out_hbm,
                     idx_v, a_v, b_v, w0_v, w1_v, sem):
    wid = lax.axis_index("s") * _NC + lax.axis_index("c")
    per_w = N // _NW
    base = wid * per_w
    for c in range(per_w // _CCH):
        off = base + c * _CCH
        pltpu.sync_copy(s0_hbm.at[pl.ds(off, _CCH)], idx_v)
        pltpu.async_copy(os_hbm.at[idx_v], a_v, sem).wait()
        pltpu.sync_copy(s1_hbm.at[pl.ds(off, _CCH)], idx_v)
        pltpu.async_copy(os_hbm.at[idx_v], b_v, sem).wait()
        pltpu.sync_copy(w0_hbm.at[pl.ds(off, _CCH)], w0_v)
        pltpu.sync_copy(w1_hbm.at[pl.ds(off, _CCH)], w1_v)

        def combine_row(r, carry):
            wa = w0_v[r, :]
            wb = w1_v[r, :]
            for j in range(D // 16):
                a_v[r, pl.ds(j * 16, 16)] = (wa * a_v[r, pl.ds(j * 16, 16)]
                                             + wb * b_v[r, pl.ds(j * 16, 16)])
            return carry

        lax.fori_loop(0, _CCH, combine_row, 0)
        pltpu.sync_copy(a_v, out_hbm.at[pl.ds(off, _CCH)])


def _sc_combine(slot0, slot1, w0r, w1r, out_sorted):
    mesh = plsc.VectorSubcoreMesh(core_axis_name="c", subcore_axis_name="s")
    fn = functools.partial(
        pl.kernel, mesh=mesh,
        out_type=jax.ShapeDtypeStruct((N, D), jnp.float32),
        scratch_types=[
            pltpu.VMEM((_CCH,), jnp.int32),
            pltpu.VMEM((_CCH, D), jnp.float32),
            pltpu.VMEM((_CCH, D), jnp.float32),
            pltpu.VMEM((_CCH, 16), jnp.float32),
            pltpu.VMEM((_CCH, 16), jnp.float32),
            pltpu.SemaphoreType.DMA,
        ],
    )(_sc_combine_body)
    return fn(slot0, slot1, w0r, w1r, out_sorted)


def kernel(x, Wg, W1, W2):
    Bb, Tt, C = x.shape
    x_flat = x.reshape(N, D)
    slot0, slot1, w0r, w1r, block_expert = _routing(x_flat, Wg)
    x_sorted = _sc_scatter(slot0, slot1, x_flat)
    out_sorted = _ffn(block_expert, x_sorted, W1, W2)
    out = _sc_combine(slot0, slot1, w0r, w1r, out_sorted)
    return out.reshape(Bb, Tt, C)
